# Initial kernel scaffold; baseline (speedup 1.0000x reference)
import jax
import jax.numpy as jnp
from jax.experimental import pallas as pl


def kernel(input_ids, emb, W, b):
    raise NotImplementedError("write your pallas kernel here")


# speedup vs baseline: 2.9083x; 2.9083x over previous
import functools

import jax
import jax.numpy as jnp
from jax import lax
from jax.experimental import pallas as pl
from jax.experimental.pallas import tpu as pltpu
from jax.experimental.pallas import tpu_sc as plsc

VOCAB = 100
DIM = 8
B_TOTAL = 16384 * 200

NUM_CORES = 2
NUM_SUBCORES = 16
NUM_WORKERS = NUM_CORES * NUM_SUBCORES
IDS_PER_WORKER = B_TOTAL // NUM_WORKERS
CHUNK = 4096
STEPS = IDS_PER_WORKER // CHUNK


def _table_body(emb_ref, w_ref, b_ref, table_ref):
    table_ref[...] = (
        jnp.dot(emb_ref[...], w_ref[...].T, preferred_element_type=jnp.float32)
        + b_ref[...]
    )


_table_call = pl.pallas_call(
    _table_body,
    out_shape=jax.ShapeDtypeStruct((VOCAB, DIM), jnp.float32),
)

_sc_mesh = plsc.VectorSubcoreMesh(core_axis_name="c", subcore_axis_name="s")


@functools.partial(
    pl.kernel,
    mesh=_sc_mesh,
    compiler_params=pltpu.CompilerParams(use_tc_tiling_on_sc=False),
    out_type=jax.ShapeDtypeStruct((B_TOTAL, DIM), jnp.float32),
    scratch_types=[
        pltpu.VMEM((CHUNK,), jnp.int32),
        pltpu.VMEM((CHUNK, DIM), jnp.float32),
        pltpu.SemaphoreType.DMA,
    ],
)
def _gather_kernel(table_hbm, idx_hbm, out_hbm, idx_v, rows_v, sem):
    wid = lax.axis_index("s") * NUM_CORES + lax.axis_index("c")
    base = wid * IDS_PER_WORKER

    def body(i, carry):
        off = base + i * CHUNK
        pltpu.sync_copy(idx_hbm.at[pl.ds(off, CHUNK)], idx_v)
        pltpu.async_copy(table_hbm.at[idx_v], rows_v, sem).wait()
        pltpu.sync_copy(rows_v, out_hbm.at[pl.ds(off, CHUNK)])
        return carry

    lax.fori_loop(0, STEPS, body, 0)


def kernel(input_ids, emb, W, b):
    table = _table_call(emb, W, b.reshape(1, DIM))
    idx = input_ids.reshape(-1).astype(jnp.int32)
    out = _gather_kernel(table, idx)
    return out.reshape(input_ids.shape + (DIM,))


# speedup vs baseline: 6.5157x; 2.2404x over previous
import functools

import jax
import jax.numpy as jnp
from jax import lax
from jax.experimental import pallas as pl
from jax.experimental.pallas import tpu as pltpu
from jax.experimental.pallas import tpu_sc as plsc

VOCAB = 100
DIM = 8
B_TOTAL = 16384 * 200

NUM_CORES = 2
NUM_SUBCORES = 16
NUM_WORKERS = NUM_CORES * NUM_SUBCORES
IDS_PER_WORKER = B_TOTAL // NUM_WORKERS
CHUNK = 4096
STEPS = IDS_PER_WORKER // CHUNK


def _table_body(emb_ref, w_ref, b_ref, table_ref):
    table_ref[...] = (
        jnp.dot(emb_ref[...], w_ref[...].T, preferred_element_type=jnp.float32)
        + b_ref[...]
    )


_table_call = pl.pallas_call(
    _table_body,
    out_shape=jax.ShapeDtypeStruct((VOCAB, DIM), jnp.float32),
)

_sc_mesh = plsc.VectorSubcoreMesh(core_axis_name="c", subcore_axis_name="s")


@functools.partial(
    pl.kernel,
    mesh=_sc_mesh,
    compiler_params=pltpu.CompilerParams(use_tc_tiling_on_sc=False),
    out_type=jax.ShapeDtypeStruct((B_TOTAL, DIM), jnp.float32),
    scratch_types=[
        pltpu.VMEM_SHARED((VOCAB, DIM), jnp.float32),
        pltpu.VMEM((CHUNK,), jnp.int32),
        pltpu.VMEM((CHUNK, DIM), jnp.float32),
        pltpu.SemaphoreType.DMA,
    ],
)
def _gather_kernel(table_hbm, idx_hbm, out_hbm, table_sh, idx_v, rows_v, sem):
    sid = lax.axis_index("s")
    wid = sid * NUM_CORES + lax.axis_index("c")
    base = wid * IDS_PER_WORKER
    @pl.when(sid == 0)
    def _():
        pltpu.sync_copy(table_hbm, table_sh)

    plsc.subcore_barrier()

    def body(i, carry):
        off = base + i * CHUNK
        pltpu.sync_copy(idx_hbm.at[pl.ds(off, CHUNK)], idx_v)
        pltpu.async_copy(table_sh.at[idx_v], rows_v, sem).wait()
        pltpu.sync_copy(rows_v, out_hbm.at[pl.ds(off, CHUNK)])
        return carry

    lax.fori_loop(0, STEPS, body, 0)


def kernel(input_ids, emb, W, b):
    table = _table_call(emb, W, b.reshape(1, DIM))
    idx = input_ids.reshape(-1).astype(jnp.int32)
    out = _gather_kernel(table, idx)
    return out.reshape(input_ids.shape + (DIM,))
